# baseline (speedup 1.0000x reference)
import functools

import jax
import jax.numpy as jnp
import numpy as np
from jax.experimental import pallas as pl
from jax.experimental.pallas import tpu as pltpu

B, T, IN = 8, 1024, 80
D, M, H, L = 512, 512, 8, 2
E = D // H

_NEG = -1e30
_LOG2E = 1.4426950408889634


def _pe_table(t, d):
    inv = 10000.0 ** np.arange(0.0, 1.0, 2.0 / d, dtype=np.float32)
    ang = np.arange(t, dtype=np.float32)[:, None] / inv[None, :]
    return np.stack([np.sin(ang), np.cos(ang)], -1).reshape(t, d)


_PE = _pe_table(T, D).astype(np.float32)


def _ln(x, g, b):
    m = jnp.mean(x, -1, keepdims=True)
    xc = x - m
    v = jnp.mean(xc * xc, -1, keepdims=True)
    return xc * jax.lax.rsqrt(v + 1e-5) * g + b


def _bf(x):
    return x.astype(jnp.bfloat16)


def _dot(a, w):
    return jnp.dot(a, w[...], preferred_element_type=jnp.float32)


def _enc_stage(seqlen, load_x, dst, pe_ref, w, h_s, q_s, k_s, v_s, tr):
    (w1h, b1h, w2h, b2h, wq1, bq1, wqc, bqc, wk1, bk1, wkc, bkc,
     wv1, bv1, wvc, bvc, wa, ba, wf1, bf1, wf2, bf2, g_ref, beta_ref) = w
    for r in range(0, T, tr):
        sl = slice(r, r + tr)
        x_t = _bf(load_x(sl))
        t1 = _bf(jnp.tanh(_dot(x_t, w1h) + b1h[...]))
        h_t = _dot(t1, w2h) + b2h[...] + pe_ref[sl, :]
        h_s[sl, :] = h_t
        hb = _bf(h_t)
        for w1, b1, wc, bc, dd in ((wq1, bq1, wqc, bqc, q_s),
                                   (wk1, bk1, wkc, bkc, k_s),
                                   (wv1, bv1, wvc, bvc, v_s)):
            u1 = _bf(jnp.tanh(_dot(hb, w1) + b1[...]))
            pr = _bf(_dot(u1, wc) + bc[...])
            for hh in range(H):
                dd[hh, sl, :] = pr[:, hh * E:(hh + 1) * E]
    madd = jnp.where(
        jax.lax.broadcasted_iota(jnp.int32, (1, T), 1) >= seqlen, _NEG, 0.0)
    for r in range(0, T, tr):
        sl = slice(r, r + tr)
        atts = []
        for hh in range(H):
            qh = q_s[hh, sl, :]
            kh = k_s[hh]
            s = jax.lax.dot_general(
                qh, kh, (((1,), (1,)), ((), ())),
                preferred_element_type=jnp.float32)
            s = s + madd
            mx = jnp.max(s, axis=-1, keepdims=True)
            p = jnp.exp2(s - mx)
            den = jnp.sum(p, axis=-1, keepdims=True)
            atts.append(_bf(_dot(_bf(p), v_s.at[hh]) / den))
        att = jnp.concatenate(atts, axis=-1)
        acc = _dot(att, wa) + ba[...]
        x2 = _ln(h_s[sl, :] + acc, g_ref[...], beta_ref[...])
        f1 = _bf(jnp.maximum(_dot(_bf(x2), wf1) + bf1[...], 0.0))
        f2 = _dot(f1, wf2) + bf2[...]
        dst[sl, :] = _ln(x2 + f2, g_ref[...], beta_ref[...])


def _head_stage(seqlen, load_x, out_ref, w, feat_s, logit_s, tr):
    wf1, bf1, wfc, bfc, ww1, bw1, wwc, bwc, wlast, blast = w
    for r in range(0, T, tr):
        sl = slice(r, r + tr)
        hb = _bf(load_x(sl))
        u1 = _bf(jnp.tanh(_dot(hb, wf1) + bf1[...]))
        feat_s[sl, :] = _dot(u1, wfc) + bfc[...]
        u2 = _bf(jnp.tanh(_dot(hb, ww1) + bw1[...]))
        logit_s[sl, :] = _dot(u2, wwc) + bwc[...]
    lg = logit_s[...]
    lg = jnp.where(
        jax.lax.broadcasted_iota(jnp.int32, (T, D), 0) >= seqlen, _NEG, lg)
    mx = jnp.max(lg, axis=0, keepdims=True)
    p = jnp.exp(lg - mx)
    den = jnp.sum(p, axis=0, keepdims=True)
    pooled = jnp.sum(p * feat_s[...], axis=0, keepdims=True) / den
    out_ref[0] = _dot(_bf(pooled), wlast) + blast[...]


_N_LAYER_ARGS = 24
_N_HEAD_ARGS = 10


def _mega_body(lens_ref, x_ref, pe_ref, *args, tr):
    w0 = args[:_N_LAYER_ARGS]
    w1 = args[_N_LAYER_ARGS:2 * _N_LAYER_ARGS]
    wh = args[2 * _N_LAYER_ARGS:2 * _N_LAYER_ARGS + _N_HEAD_ARGS]
    (out_ref, cur_s, h_s, q_s, k_s, v_s, feat_s, logit_s) = \
        args[2 * _N_LAYER_ARGS + _N_HEAD_ARGS:]
    b = pl.program_id(0)
    seqlen = lens_ref[b]
    _enc_stage(seqlen, lambda sl: x_ref[0, sl, :], cur_s, pe_ref, w0,
               h_s, q_s, k_s, v_s, tr)
    _enc_stage(seqlen, lambda sl: cur_s[sl, :], cur_s, pe_ref, w1,
               h_s, q_s, k_s, v_s, tr)
    _head_stage(seqlen, lambda sl: cur_s[sl, :], out_ref, wh,
                feat_s, logit_s, tr)


def _lin_w(p):
    return p["w"].T


def _fold(mlp, split_w2d, split_b):
    ws = split_w2d.T
    wc = mlp["l2"]["w"].T @ ws
    bc = mlp["l2"]["b"][None, :] @ ws + split_b[None, :]
    return wc, bc


def _prep_layer(p, scale_q):
    th, ff = p["to_hidden"], p["ff"]
    wqc, bqc = _fold(p["q_mlp"], p["q_split"]["w"].reshape(H * E, D),
                     p["q_split"]["b"].reshape(H * E))
    wkc, bkc = _fold(p["k_mlp"], p["k_split"]["w"].reshape(H * E, D),
                     p["k_split"]["b"].reshape(H * E))
    wvc, bvc = _fold(p["v_mlp"], p["v_split"]["w"].reshape(H * E, D),
                     p["v_split"]["b"].reshape(H * E))
    wqc, bqc = wqc * (scale_q * _LOG2E), bqc * (scale_q * _LOG2E)
    return [
        _bf(_lin_w(th["l1"])), th["l1"]["b"][None],
        _bf(_lin_w(th["l2"])), th["l2"]["b"][None],
        _bf(_lin_w(p["q_mlp"]["l1"])), p["q_mlp"]["l1"]["b"][None],
        _bf(wqc), bqc,
        _bf(_lin_w(p["k_mlp"]["l1"])), p["k_mlp"]["l1"]["b"][None],
        _bf(wkc), bkc,
        _bf(_lin_w(p["v_mlp"]["l1"])), p["v_mlp"]["l1"]["b"][None],
        _bf(wvc), bvc,
        _bf(_lin_w(p["h2a"])), p["h2a"]["b"][None],
        _bf(_lin_w(ff["l1"])), ff["l1"]["b"][None],
        _bf(_lin_w(ff["l2"])), ff["l2"]["b"][None],
        p["ln_g"][None], p["ln_b"][None],
    ]


def _prep_head(params):
    wfc, bfc = _fold(params["ff_mlp"],
                     params["ff_split"]["w"].reshape(H * E, D),
                     params["ff_split"]["b"].reshape(H * E))
    wwc_s, bwc_s = _fold(params["fw_mlp"], params["fw_split"]["w"][:, 0, :],
                         params["fw_split"]["b"][:, 0])
    wwc = jnp.repeat(wwc_s, E, axis=1)
    bwc = jnp.repeat(bwc_s, E, axis=1)
    return [
        _bf(_lin_w(params["ff_mlp"]["l1"])), params["ff_mlp"]["l1"]["b"][None],
        _bf(wfc), bfc,
        _bf(_lin_w(params["fw_mlp"]["l1"])), params["fw_mlp"]["l1"]["b"][None],
        _bf(wwc), bwc,
        _bf(_lin_w(params["last"])), params["last"]["b"][None],
    ]


def _full_spec(shape):
    nd = len(shape)
    return pl.BlockSpec(shape, lambda b: (0,) * nd)


def kernel(x, lengths, params, interpret=False, tr=256):
    pe = jnp.asarray(_PE)
    lens = lengths.astype(jnp.int32)
    wargs = _prep_layer(params["layers"][0], E ** -0.5)
    wargs += _prep_layer(params["layers"][1], E ** -0.5)
    wargs += _prep_head(params)
    in_specs = [pl.BlockSpec(memory_space=pltpu.SMEM),
                pl.BlockSpec((1, T, IN), lambda b: (b, 0, 0)),
                _full_spec((T, D))]
    in_specs += [_full_spec(w.shape) for w in wargs]
    out = pl.pallas_call(
        functools.partial(_mega_body, tr=tr),
        out_shape=jax.ShapeDtypeStruct((B, 1, D), jnp.float32),
        grid=(B,),
        in_specs=in_specs,
        out_specs=pl.BlockSpec((1, 1, D), lambda b: (b, 0, 0)),
        scratch_shapes=[
            pltpu.VMEM((T, D), jnp.float32),
            pltpu.VMEM((T, D), jnp.float32),
            pltpu.VMEM((H, T, E), jnp.bfloat16),
            pltpu.VMEM((H, T, E), jnp.bfloat16),
            pltpu.VMEM((H, T, E), jnp.bfloat16),
            pltpu.VMEM((T, D), jnp.float32),
            pltpu.VMEM((T, D), jnp.float32),
        ],
        compiler_params=pltpu.CompilerParams(
            dimension_semantics=("parallel",),
            vmem_limit_bytes=100 * 1024 * 1024,
        ),
        name="enc_pool_fused",
        interpret=interpret,
    )(lens, x, pe, *wargs)
    return out.reshape(B, D)


# speedup vs baseline: 1.1581x; 1.1581x over previous
import functools

import jax
import jax.numpy as jnp
import numpy as np
from jax.experimental import pallas as pl
from jax.experimental.pallas import tpu as pltpu

B, T, IN = 8, 1024, 80
D, M, H, L = 512, 512, 8, 2
E = D // H

_NEG = -1e30
_LOG2E = 1.4426950408889634


def _pe_table(t, d):
    inv = 10000.0 ** np.arange(0.0, 1.0, 2.0 / d, dtype=np.float32)
    ang = np.arange(t, dtype=np.float32)[:, None] / inv[None, :]
    return np.stack([np.sin(ang), np.cos(ang)], -1).reshape(t, d)


_PE = _pe_table(T, D).astype(np.float32)


def _ln(x, g, b):
    m = jnp.mean(x, -1, keepdims=True)
    xc = x - m
    v = jnp.mean(xc * xc, -1, keepdims=True)
    return xc * jax.lax.rsqrt(v + 1e-5) * g + b


def _bf(x):
    return x.astype(jnp.bfloat16)


def _dot(a, w):
    return jnp.dot(a, w[...], preferred_element_type=jnp.float32)


def _layer_body(lens_ref, x_ref, pe_ref,
                w1h, b1h, w2h, b2h,
                wq1, bq1, wqc, bqc,
                wk1, bk1, wkc, bkc,
                wv1, bv1, wvc, bvc,
                wa, ba, wf1, bf1, wf2, bf2,
                g_ref, beta_ref,
                out_ref, h_s, q_s, k_s, v_s, *, tr):
    b = pl.program_id(0)
    seqlen = lens_ref[b]
    for r in range(0, T, tr):
        sl = slice(r, r + tr)
        x_t = _bf(x_ref[0, sl, :])
        t1 = _bf(jnp.tanh(_dot(x_t, w1h) + b1h[...]))
        h_t = _dot(t1, w2h) + b2h[...] + pe_ref[sl, :]
        h_s[sl, :] = h_t
        hb = _bf(h_t)
        for w1, b1, wc, bc, dd in ((wq1, bq1, wqc, bqc, q_s),
                                   (wk1, bk1, wkc, bkc, k_s),
                                   (wv1, bv1, wvc, bvc, v_s)):
            u1 = _bf(jnp.tanh(_dot(hb, w1) + b1[...]))
            pr = _bf(_dot(u1, wc) + bc[...])
            for hh in range(H):
                dd[hh, sl, :] = pr[:, hh * E:(hh + 1) * E]
    madd = jnp.where(
        jax.lax.broadcasted_iota(jnp.int32, (1, T), 1) >= seqlen, _NEG, 0.0)
    for r in range(0, T, tr):
        sl = slice(r, r + tr)
        atts = []
        for hh in range(H):
            qh = q_s[hh, sl, :]
            kh = k_s[hh]
            s = jax.lax.dot_general(
                qh, kh, (((1,), (1,)), ((), ())),
                preferred_element_type=jnp.float32)
            s = s + madd
            mx = jnp.max(s, axis=-1, keepdims=True)
            p = jnp.exp2(s - mx)
            den = jnp.sum(p, axis=-1, keepdims=True)
            atts.append(_bf(_dot(_bf(p), v_s.at[hh]) / den))
        att = jnp.concatenate(atts, axis=-1)
        acc = _dot(att, wa) + ba[...]
        x2 = _ln(h_s[sl, :] + acc, g_ref[...], beta_ref[...])
        f1 = _bf(jnp.maximum(_dot(_bf(x2), wf1) + bf1[...], 0.0))
        f2 = _dot(f1, wf2) + bf2[...]
        out_ref[0, sl, :] = _ln(x2 + f2, g_ref[...], beta_ref[...])


def _head_body(lens_ref, x_ref,
               wf1, bf1, wfc, bfc, ww1, bw1, wwc, bwc, wlast, blast,
               out_ref, feat_s, logit_s, *, tr):
    b = pl.program_id(0)
    seqlen = lens_ref[b]
    mx = jnp.full((1, D), _NEG, jnp.float32)
    for r in range(0, T, tr):
        sl = slice(r, r + tr)
        hb = _bf(x_ref[0, sl, :])
        u1 = _bf(jnp.tanh(_dot(hb, wf1) + bf1[...]))
        feat_s[sl, :] = _dot(u1, wfc) + bfc[...]
        u2 = _bf(jnp.tanh(_dot(hb, ww1) + bw1[...]))
        lg = _dot(u2, wwc) + bwc[...]
        lg = jnp.where(
            jax.lax.broadcasted_iota(jnp.int32, (tr, D), 0) + r >= seqlen,
            _NEG, lg)
        logit_s[sl, :] = lg
        mx = jnp.maximum(mx, jnp.max(lg, axis=0, keepdims=True))
    num = jnp.zeros((1, D), jnp.float32)
    den = jnp.zeros((1, D), jnp.float32)
    for r in range(0, T, tr):
        sl = slice(r, r + tr)
        p = jnp.exp2(logit_s[sl, :] - mx)
        num = num + jnp.sum(p * feat_s[sl, :], axis=0, keepdims=True)
        den = den + jnp.sum(p, axis=0, keepdims=True)
    pooled = num / den
    out_ref[0] = _dot(_bf(pooled), wlast) + blast[...]


def _lin_w(p):
    return p["w"].T


def _fold(mlp, split_w2d, split_b):
    ws = split_w2d.T
    wc = mlp["l2"]["w"].T @ ws
    bc = mlp["l2"]["b"][None, :] @ ws + split_b[None, :]
    return wc, bc


def _prep_layer(p, scale_q):
    th, ff = p["to_hidden"], p["ff"]
    wqc, bqc = _fold(p["q_mlp"], p["q_split"]["w"].reshape(H * E, D),
                     p["q_split"]["b"].reshape(H * E))
    wkc, bkc = _fold(p["k_mlp"], p["k_split"]["w"].reshape(H * E, D),
                     p["k_split"]["b"].reshape(H * E))
    wvc, bvc = _fold(p["v_mlp"], p["v_split"]["w"].reshape(H * E, D),
                     p["v_split"]["b"].reshape(H * E))
    wqc, bqc = wqc * (scale_q * _LOG2E), bqc * (scale_q * _LOG2E)
    return [
        _bf(_lin_w(th["l1"])), th["l1"]["b"][None],
        _bf(_lin_w(th["l2"])), th["l2"]["b"][None],
        _bf(_lin_w(p["q_mlp"]["l1"])), p["q_mlp"]["l1"]["b"][None],
        _bf(wqc), bqc,
        _bf(_lin_w(p["k_mlp"]["l1"])), p["k_mlp"]["l1"]["b"][None],
        _bf(wkc), bkc,
        _bf(_lin_w(p["v_mlp"]["l1"])), p["v_mlp"]["l1"]["b"][None],
        _bf(wvc), bvc,
        _bf(_lin_w(p["h2a"])), p["h2a"]["b"][None],
        _bf(_lin_w(ff["l1"])), ff["l1"]["b"][None],
        _bf(_lin_w(ff["l2"])), ff["l2"]["b"][None],
        p["ln_g"][None], p["ln_b"][None],
    ]


def _prep_head(params):
    wfc, bfc = _fold(params["ff_mlp"],
                     params["ff_split"]["w"].reshape(H * E, D),
                     params["ff_split"]["b"].reshape(H * E))
    wwc_s, bwc_s = _fold(params["fw_mlp"], params["fw_split"]["w"][:, 0, :],
                         params["fw_split"]["b"][:, 0])
    wwc = jnp.repeat(wwc_s * _LOG2E, E, axis=1)
    bwc = jnp.repeat(bwc_s * _LOG2E, E, axis=1)
    return [
        _bf(_lin_w(params["ff_mlp"]["l1"])), params["ff_mlp"]["l1"]["b"][None],
        _bf(wfc), bfc,
        _bf(_lin_w(params["fw_mlp"]["l1"])), params["fw_mlp"]["l1"]["b"][None],
        _bf(wwc), bwc,
        _bf(_lin_w(params["last"])), params["last"]["b"][None],
    ]


def _full_spec(shape):
    nd = len(shape)
    return pl.BlockSpec(shape, lambda b: (0,) * nd)


_CPARAMS = pltpu.CompilerParams(
    dimension_semantics=("parallel",),
    vmem_limit_bytes=100 * 1024 * 1024,
)


def _layer_call(x, pe, lens, wargs, ci, tr, interpret):
    in_specs = [pl.BlockSpec(memory_space=pltpu.SMEM),
                pl.BlockSpec((1, T, ci), lambda b: (b, 0, 0)),
                _full_spec((T, D))]
    in_specs += [_full_spec(w.shape) for w in wargs]
    return pl.pallas_call(
        functools.partial(_layer_body, tr=tr),
        out_shape=jax.ShapeDtypeStruct((B, T, D), jnp.float32),
        grid=(B,),
        in_specs=in_specs,
        out_specs=pl.BlockSpec((1, T, D), lambda b: (b, 0, 0)),
        scratch_shapes=[
            pltpu.VMEM((T, D), jnp.float32),
            pltpu.VMEM((H, T, E), jnp.bfloat16),
            pltpu.VMEM((H, T, E), jnp.bfloat16),
            pltpu.VMEM((H, T, E), jnp.bfloat16),
        ],
        compiler_params=_CPARAMS,
        name="enc_layer",
        interpret=interpret,
    )(lens, x, pe, *wargs)


def _head_call(x, lens, wargs, tr, interpret):
    in_specs = [pl.BlockSpec(memory_space=pltpu.SMEM),
                pl.BlockSpec((1, T, D), lambda b: (b, 0, 0))]
    in_specs += [_full_spec(w.shape) for w in wargs]
    return pl.pallas_call(
        functools.partial(_head_body, tr=tr),
        out_shape=jax.ShapeDtypeStruct((B, 1, D), jnp.float32),
        grid=(B,),
        in_specs=in_specs,
        out_specs=pl.BlockSpec((1, 1, D), lambda b: (b, 0, 0)),
        scratch_shapes=[
            pltpu.VMEM((T, D), jnp.float32),
            pltpu.VMEM((T, D), jnp.float32),
        ],
        compiler_params=_CPARAMS,
        name="attn_pool_head",
        interpret=interpret,
    )(lens, x, *wargs)


def kernel(x, lengths, params, interpret=False, tr=256):
    pe = jnp.asarray(_PE)
    lens = lengths.astype(jnp.int32)
    h = x
    for li, p in enumerate(params["layers"]):
        ci = IN if li == 0 else D
        wargs = _prep_layer(p, E ** -0.5)
        h = _layer_call(h, pe, lens, wargs, ci, tr, interpret)
    hargs = _prep_head(params)
    return _head_call(h, lens, hargs, tr, interpret).reshape(B, D)


# speedup vs baseline: 1.6770x; 1.4481x over previous
import functools

import jax
import jax.numpy as jnp
import numpy as np
from jax.experimental import pallas as pl
from jax.experimental.pallas import tpu as pltpu

B, T, IN = 8, 1024, 80
D, M, H, L = 512, 512, 8, 2
E = D // H

_NEG = -1e30
_LOG2E = 1.4426950408889634


def _pe_table(t, d):
    inv = 10000.0 ** np.arange(0.0, 1.0, 2.0 / d, dtype=np.float32)
    ang = np.arange(t, dtype=np.float32)[:, None] / inv[None, :]
    return np.stack([np.sin(ang), np.cos(ang)], -1).reshape(t, d)


_PE = _pe_table(T, D).astype(np.float32)


def _ln(x, g, b):
    m = jnp.mean(x, -1, keepdims=True)
    xc = x - m
    v = jnp.mean(xc * xc, -1, keepdims=True)
    return xc * jax.lax.rsqrt(v + 1e-5) * g + b


def _bf(x):
    return x.astype(jnp.bfloat16)


def _dot(a, w):
    return jnp.dot(a, w[...], preferred_element_type=jnp.float32)


def _layer_body(lens_ref, x_ref, pe_ref,
                w1h, b1h, w2h, b2h,
                wq1, bq1, wqc, bqc,
                wk1, bk1, wkc, bkc,
                wv1, bv1, wvc, bvc,
                wa, ba, wf1, bf1, wf2, bf2,
                g_ref, beta_ref,
                out_ref, h_s, q_s, k_s, v_s, *, tr):
    b = pl.program_id(0)
    seqlen = lens_ref[b]
    for r in range(0, T, tr):
        sl = slice(r, r + tr)
        x_t = _bf(x_ref[0, sl, :])
        t1 = _bf(jnp.tanh(_dot(x_t, w1h) + b1h[...]))
        h_t = _dot(t1, w2h) + b2h[...] + pe_ref[sl, :]
        h_s[sl, :] = h_t
        hb = _bf(h_t)
        for w1, b1, wc, bc, dd in ((wq1, bq1, wqc, bqc, q_s),
                                   (wk1, bk1, wkc, bkc, k_s),
                                   (wv1, bv1, wvc, bvc, v_s)):
            u1 = _bf(jnp.tanh(_dot(hb, w1) + b1[...]))
            pr = _bf(_dot(u1, wc) + bc[...])
            for hh in range(H):
                dd[hh, sl, :] = pr[:, hh * E:(hh + 1) * E]
    madd = jnp.where(
        jax.lax.broadcasted_iota(jnp.int32, (1, T), 1) >= seqlen, _NEG, 0.0)
    for r in range(0, T, tr):
        sl = slice(r, r + tr)
        atts = []
        for hh in range(H):
            qh = q_s[hh, sl, :]
            kh = k_s[hh]
            s = jax.lax.dot_general(
                qh, kh, (((1,), (1,)), ((), ())),
                preferred_element_type=jnp.float32)
            s = s + madd
            mx = jnp.max(s, axis=-1, keepdims=True)
            p = jnp.exp2(s - mx)
            den = jnp.sum(p, axis=-1, keepdims=True)
            atts.append(_bf(_dot(_bf(p), v_s.at[hh]) / den))
        att = jnp.concatenate(atts, axis=-1)
        acc = _dot(att, wa) + ba[...]
        x2 = _ln(h_s[sl, :] + acc, g_ref[...], beta_ref[...])
        f1 = _bf(jnp.maximum(_dot(_bf(x2), wf1) + bf1[...], 0.0))
        f2 = _dot(f1, wf2) + bf2[...]
        out_ref[0, sl, :] = _ln(x2 + f2, g_ref[...], beta_ref[...])


def _head_body(lens_ref, x_ref,
               wf1, bf1, wfc, bfc, ww1, bw1, wwc, bwc, wlast, blast,
               out_ref, feat_s, logit_s, *, tr):
    b = pl.program_id(0)
    seqlen = lens_ref[b]
    mx = jnp.full((1, D), _NEG, jnp.float32)
    for r in range(0, T, tr):
        sl = slice(r, r + tr)
        hb = _bf(x_ref[0, sl, :])
        u1 = _bf(jnp.tanh(_dot(hb, wf1) + bf1[...]))
        feat_s[sl, :] = _dot(u1, wfc) + bfc[...]
        u2 = _bf(jnp.tanh(_dot(hb, ww1) + bw1[...]))
        lg = _dot(u2, wwc) + bwc[...]
        lg = jnp.where(
            jax.lax.broadcasted_iota(jnp.int32, (tr, D), 0) + r >= seqlen,
            _NEG, lg)
        logit_s[sl, :] = lg
        mx = jnp.maximum(mx, jnp.max(lg, axis=0, keepdims=True))
    num = jnp.zeros((1, D), jnp.float32)
    den = jnp.zeros((1, D), jnp.float32)
    for r in range(0, T, tr):
        sl = slice(r, r + tr)
        p = jnp.exp2(logit_s[sl, :] - mx)
        num = num + jnp.sum(p * feat_s[sl, :], axis=0, keepdims=True)
        den = den + jnp.sum(p, axis=0, keepdims=True)
    pooled = num / den
    out_ref[0] = _dot(_bf(pooled), wlast) + blast[...]


def _lin_w(p):
    return p["w"].T


def _fold(mlp, split_w2d, split_b):
    ws = split_w2d.T
    wc = mlp["l2"]["w"].T @ ws
    bc = mlp["l2"]["b"][None, :] @ ws + split_b[None, :]
    return wc, bc


def _prep_layer(p, scale_q):
    th, ff = p["to_hidden"], p["ff"]
    wqc, bqc = _fold(p["q_mlp"], p["q_split"]["w"].reshape(H * E, D),
                     p["q_split"]["b"].reshape(H * E))
    wkc, bkc = _fold(p["k_mlp"], p["k_split"]["w"].reshape(H * E, D),
                     p["k_split"]["b"].reshape(H * E))
    wvc, bvc = _fold(p["v_mlp"], p["v_split"]["w"].reshape(H * E, D),
                     p["v_split"]["b"].reshape(H * E))
    wqc, bqc = wqc * (scale_q * _LOG2E), bqc * (scale_q * _LOG2E)
    return [
        _bf(_lin_w(th["l1"])), th["l1"]["b"][None],
        _bf(_lin_w(th["l2"])), th["l2"]["b"][None],
        _bf(_lin_w(p["q_mlp"]["l1"])), p["q_mlp"]["l1"]["b"][None],
        _bf(wqc), bqc,
        _bf(_lin_w(p["k_mlp"]["l1"])), p["k_mlp"]["l1"]["b"][None],
        _bf(wkc), bkc,
        _bf(_lin_w(p["v_mlp"]["l1"])), p["v_mlp"]["l1"]["b"][None],
        _bf(wvc), bvc,
        _bf(_lin_w(p["h2a"])), p["h2a"]["b"][None],
        _bf(_lin_w(ff["l1"])), ff["l1"]["b"][None],
        _bf(_lin_w(ff["l2"])), ff["l2"]["b"][None],
        p["ln_g"][None], p["ln_b"][None],
    ]


def _prep_head(params):
    wfc, bfc = _fold(params["ff_mlp"],
                     params["ff_split"]["w"].reshape(H * E, D),
                     params["ff_split"]["b"].reshape(H * E))
    wwc_s, bwc_s = _fold(params["fw_mlp"], params["fw_split"]["w"][:, 0, :],
                         params["fw_split"]["b"][:, 0])
    wwc = jnp.repeat(wwc_s * _LOG2E, E, axis=1)
    bwc = jnp.repeat(bwc_s * _LOG2E, E, axis=1)
    return [
        _bf(_lin_w(params["ff_mlp"]["l1"])), params["ff_mlp"]["l1"]["b"][None],
        _bf(wfc), bfc,
        _bf(_lin_w(params["fw_mlp"]["l1"])), params["fw_mlp"]["l1"]["b"][None],
        _bf(wwc), bwc,
        _bf(_lin_w(params["last"])), params["last"]["b"][None],
    ]


def _full_spec(shape):
    nd = len(shape)
    return pl.BlockSpec(shape, lambda b: (0,) * nd)


_CPARAMS = pltpu.CompilerParams(
    dimension_semantics=("parallel",),
    vmem_limit_bytes=100 * 1024 * 1024,
)


def _layer_call(x, pe, lens, wargs, ci, tr, interpret):
    in_specs = [pl.BlockSpec(memory_space=pltpu.SMEM),
                pl.BlockSpec((1, T, ci), lambda b: (b, 0, 0)),
                _full_spec((T, D))]
    in_specs += [_full_spec(w.shape) for w in wargs]
    return pl.pallas_call(
        functools.partial(_layer_body, tr=tr),
        out_shape=jax.ShapeDtypeStruct((B, T, D), jnp.float32),
        grid=(B,),
        in_specs=in_specs,
        out_specs=pl.BlockSpec((1, T, D), lambda b: (b, 0, 0)),
        scratch_shapes=[
            pltpu.VMEM((T, D), jnp.float32),
            pltpu.VMEM((H, T, E), jnp.bfloat16),
            pltpu.VMEM((H, T, E), jnp.bfloat16),
            pltpu.VMEM((H, T, E), jnp.bfloat16),
        ],
        compiler_params=_CPARAMS,
        name="enc_layer",
        interpret=interpret,
    )(lens, x, pe, *wargs)


def _head_call(x, lens, wargs, tr, interpret):
    in_specs = [pl.BlockSpec(memory_space=pltpu.SMEM),
                pl.BlockSpec((1, T, D), lambda b: (b, 0, 0))]
    in_specs += [_full_spec(w.shape) for w in wargs]
    return pl.pallas_call(
        functools.partial(_head_body, tr=tr),
        out_shape=jax.ShapeDtypeStruct((B, 1, D), jnp.float32),
        grid=(B,),
        in_specs=in_specs,
        out_specs=pl.BlockSpec((1, 1, D), lambda b: (b, 0, 0)),
        scratch_shapes=[
            pltpu.VMEM((T, D), jnp.float32),
            pltpu.VMEM((T, D), jnp.float32),
        ],
        compiler_params=_CPARAMS,
        name="attn_pool_head",
        interpret=interpret,
    )(lens, x, *wargs)


def kernel(x, lengths, params, interpret=False, tr=512):
    pe = jnp.asarray(_PE)
    lens = lengths.astype(jnp.int32)
    h = x
    for li, p in enumerate(params["layers"]):
        ci = IN if li == 0 else D
        wargs = _prep_layer(p, E ** -0.5)
        h = _layer_call(h, pe, lens, wargs, ci, tr, interpret)
    hargs = _prep_head(params)
    return _head_call(h, lens, hargs, tr, interpret).reshape(B, D)


# speedup vs baseline: 1.7800x; 1.0614x over previous
import functools

import jax
import jax.numpy as jnp
import numpy as np
from jax.experimental import pallas as pl
from jax.experimental.pallas import tpu as pltpu

B, T, IN = 8, 1024, 80
D, M, H, L = 512, 512, 8, 2
E = D // H

_NEG = -1e30
_LOG2E = 1.4426950408889634


def _pe_table(t, d):
    inv = 10000.0 ** np.arange(0.0, 1.0, 2.0 / d, dtype=np.float32)
    ang = np.arange(t, dtype=np.float32)[:, None] / inv[None, :]
    return np.stack([np.sin(ang), np.cos(ang)], -1).reshape(t, d)


_PE = _pe_table(T, D).astype(np.float32)


def _ln(x, g, b):
    m = jnp.mean(x, -1, keepdims=True)
    xc = x - m
    v = jnp.mean(xc * xc, -1, keepdims=True)
    return xc * jax.lax.rsqrt(v + 1e-5) * g + b


def _bf(x):
    return x.astype(jnp.bfloat16)


def _dot(a, w):
    return jnp.dot(a, w[...], preferred_element_type=jnp.float32)


def _layer_body(lens_ref, x_ref, pe_ref,
                w1h, b1h, w2h, b2h,
                wq1, bq1, wqc, bqc,
                wk1, bk1, wkc, bkc,
                wv1, bv1, wvc, bvc,
                wa, ba, wf1, bf1, wf2, bf2,
                g_ref, beta_ref,
                out_ref, h_s, q_s, k_s, v_s, *, tr):
    b = pl.program_id(0)
    seqlen = lens_ref[b]
    for r in range(0, T, tr):
        sl = slice(r, r + tr)
        x_t = _bf(x_ref[0, sl, :])
        t1 = _bf(jnp.tanh(_dot(x_t, w1h) + b1h[...]))
        h_t = _dot(t1, w2h) + b2h[...] + pe_ref[sl, :]
        h_s[sl, :] = h_t
        hb = _bf(h_t)
        for w1, b1, wc, bc, dd in ((wq1, bq1, wqc, bqc, q_s),
                                   (wk1, bk1, wkc, bkc, k_s),
                                   (wv1, bv1, wvc, bvc, v_s)):
            u1 = _bf(jnp.tanh(_dot(hb, w1) + b1[...]))
            pr = _bf(_dot(u1, wc) + bc[...])
            for hh in range(H):
                dd[hh, sl, :] = pr[:, hh * E:(hh + 1) * E]
    madd = jnp.where(
        jax.lax.broadcasted_iota(jnp.int32, (1, T), 1) >= seqlen, _NEG, 0.0)
    for r in range(0, T, tr):
        sl = slice(r, r + tr)
        atts = []
        for hh in range(H):
            qh = q_s[hh, sl, :]
            kh = k_s[hh]
            s = jax.lax.dot_general(
                qh, kh, (((1,), (1,)), ((), ())),
                preferred_element_type=jnp.float32)
            s = s + madd
            mx = jnp.max(s, axis=-1, keepdims=True)
            p = jnp.exp2(s - mx)
            den = jnp.sum(p, axis=-1, keepdims=True)
            atts.append(_bf(_dot(_bf(p), v_s.at[hh]) / den))
        att = jnp.concatenate(atts, axis=-1)
        acc = _dot(att, wa) + ba[...]
        x2 = _ln(h_s[sl, :] + acc, g_ref[...], beta_ref[...])
        f1 = _bf(jnp.maximum(_dot(_bf(x2), wf1) + bf1[...], 0.0))
        f2 = _dot(f1, wf2) + bf2[...]
        out_ref[0, sl, :] = _ln(x2 + f2, g_ref[...], beta_ref[...])


def _head_body(lens_ref, x_ref,
               wf1, bf1, wfc, bfc, ww1, bw1, wwc, bwc, wlast, blast,
               out_ref, feat_s, logit_s, *, tr):
    b = pl.program_id(0)
    seqlen = lens_ref[b]
    mx = jnp.full((1, D), _NEG, jnp.float32)
    for r in range(0, T, tr):
        sl = slice(r, r + tr)
        hb = _bf(x_ref[0, sl, :])
        u1 = _bf(jnp.tanh(_dot(hb, wf1) + bf1[...]))
        feat_s[sl, :] = _dot(u1, wfc) + bfc[...]
        u2 = _bf(jnp.tanh(_dot(hb, ww1) + bw1[...]))
        lg = _dot(u2, wwc) + bwc[...]
        lg = jnp.where(
            jax.lax.broadcasted_iota(jnp.int32, (tr, D), 0) + r >= seqlen,
            _NEG, lg)
        logit_s[sl, :] = lg
        mx = jnp.maximum(mx, jnp.max(lg, axis=0, keepdims=True))
    num = jnp.zeros((1, D), jnp.float32)
    den = jnp.zeros((1, D), jnp.float32)
    for r in range(0, T, tr):
        sl = slice(r, r + tr)
        p = jnp.exp2(logit_s[sl, :] - mx)
        num = num + jnp.sum(p * feat_s[sl, :], axis=0, keepdims=True)
        den = den + jnp.sum(p, axis=0, keepdims=True)
    pooled = num / den
    out_ref[0] = _dot(_bf(pooled), wlast) + blast[...]


def _lin_w(p):
    return p["w"].T


def _fold(mlp, split_w2d, split_b):
    ws = split_w2d.T
    wc = mlp["l2"]["w"].T @ ws
    bc = mlp["l2"]["b"][None, :] @ ws + split_b[None, :]
    return wc, bc


def _prep_layer(p, scale_q):
    th, ff = p["to_hidden"], p["ff"]
    wqc, bqc = _fold(p["q_mlp"], p["q_split"]["w"].reshape(H * E, D),
                     p["q_split"]["b"].reshape(H * E))
    wkc, bkc = _fold(p["k_mlp"], p["k_split"]["w"].reshape(H * E, D),
                     p["k_split"]["b"].reshape(H * E))
    wvc, bvc = _fold(p["v_mlp"], p["v_split"]["w"].reshape(H * E, D),
                     p["v_split"]["b"].reshape(H * E))
    wqc, bqc = wqc * (scale_q * _LOG2E), bqc * (scale_q * _LOG2E)
    return [
        _bf(_lin_w(th["l1"])), th["l1"]["b"][None],
        _bf(_lin_w(th["l2"])), th["l2"]["b"][None],
        _bf(_lin_w(p["q_mlp"]["l1"])), p["q_mlp"]["l1"]["b"][None],
        _bf(wqc), bqc,
        _bf(_lin_w(p["k_mlp"]["l1"])), p["k_mlp"]["l1"]["b"][None],
        _bf(wkc), bkc,
        _bf(_lin_w(p["v_mlp"]["l1"])), p["v_mlp"]["l1"]["b"][None],
        _bf(wvc), bvc,
        _bf(_lin_w(p["h2a"])), p["h2a"]["b"][None],
        _bf(_lin_w(ff["l1"])), ff["l1"]["b"][None],
        _bf(_lin_w(ff["l2"])), ff["l2"]["b"][None],
        p["ln_g"][None], p["ln_b"][None],
    ]


def _prep_head(params):
    wfc, bfc = _fold(params["ff_mlp"],
                     params["ff_split"]["w"].reshape(H * E, D),
                     params["ff_split"]["b"].reshape(H * E))
    wwc_s, bwc_s = _fold(params["fw_mlp"], params["fw_split"]["w"][:, 0, :],
                         params["fw_split"]["b"][:, 0])
    wwc = jnp.repeat(wwc_s * _LOG2E, E, axis=1)
    bwc = jnp.repeat(bwc_s * _LOG2E, E, axis=1)
    return [
        _bf(_lin_w(params["ff_mlp"]["l1"])), params["ff_mlp"]["l1"]["b"][None],
        _bf(wfc), bfc,
        _bf(_lin_w(params["fw_mlp"]["l1"])), params["fw_mlp"]["l1"]["b"][None],
        _bf(wwc), bwc,
        _bf(_lin_w(params["last"])), params["last"]["b"][None],
    ]


def _full_spec(shape):
    nd = len(shape)
    return pl.BlockSpec(shape, lambda b: (0,) * nd)


_CPARAMS = pltpu.CompilerParams(
    dimension_semantics=("parallel",),
    vmem_limit_bytes=100 * 1024 * 1024,
)


def _layer_call(x, pe, lens, wargs, ci, tr, interpret):
    in_specs = [pl.BlockSpec(memory_space=pltpu.SMEM),
                pl.BlockSpec((1, T, ci), lambda b: (b, 0, 0)),
                _full_spec((T, D))]
    in_specs += [_full_spec(w.shape) for w in wargs]
    return pl.pallas_call(
        functools.partial(_layer_body, tr=tr),
        out_shape=jax.ShapeDtypeStruct((B, T, D), jnp.float32),
        grid=(B,),
        in_specs=in_specs,
        out_specs=pl.BlockSpec((1, T, D), lambda b: (b, 0, 0)),
        scratch_shapes=[
            pltpu.VMEM((T, D), jnp.float32),
            pltpu.VMEM((H, T, E), jnp.bfloat16),
            pltpu.VMEM((H, T, E), jnp.bfloat16),
            pltpu.VMEM((H, T, E), jnp.bfloat16),
        ],
        compiler_params=_CPARAMS,
        name="enc_layer",
        interpret=interpret,
    )(lens, x, pe, *wargs)


def _head_call(x, lens, wargs, tr, interpret):
    in_specs = [pl.BlockSpec(memory_space=pltpu.SMEM),
                pl.BlockSpec((1, T, D), lambda b: (b, 0, 0))]
    in_specs += [_full_spec(w.shape) for w in wargs]
    return pl.pallas_call(
        functools.partial(_head_body, tr=tr),
        out_shape=jax.ShapeDtypeStruct((B, 1, D), jnp.float32),
        grid=(B,),
        in_specs=in_specs,
        out_specs=pl.BlockSpec((1, 1, D), lambda b: (b, 0, 0)),
        scratch_shapes=[
            pltpu.VMEM((T, D), jnp.float32),
            pltpu.VMEM((T, D), jnp.float32),
        ],
        compiler_params=_CPARAMS,
        name="attn_pool_head",
        interpret=interpret,
    )(lens, x, *wargs)


def kernel(x, lengths, params, interpret=False, tr=1024):
    pe = jnp.asarray(_PE)
    lens = lengths.astype(jnp.int32)
    h = x
    for li, p in enumerate(params["layers"]):
        ci = IN if li == 0 else D
        wargs = _prep_layer(p, E ** -0.5)
        h = _layer_call(h, pe, lens, wargs, ci, tr, interpret)
    hargs = _prep_head(params)
    return _head_call(h, lens, hargs, tr, interpret).reshape(B, D)


# speedup vs baseline: 1.8259x; 1.0258x over previous
import functools

import jax
import jax.numpy as jnp
import numpy as np
from jax.experimental import pallas as pl
from jax.experimental.pallas import tpu as pltpu

B, T, IN = 8, 1024, 80
D, M, H, L = 512, 512, 8, 2
E = D // H

_NEG = -1e30
_LOG2E = 1.4426950408889634


def _pe_table(t, d):
    inv = 10000.0 ** np.arange(0.0, 1.0, 2.0 / d, dtype=np.float32)
    ang = np.arange(t, dtype=np.float32)[:, None] / inv[None, :]
    return np.stack([np.sin(ang), np.cos(ang)], -1).reshape(t, d)


_PE = _pe_table(T, D).astype(np.float32)


def _ln(x, g, b):
    m = jnp.mean(x, -1, keepdims=True)
    xc = x - m
    v = jnp.mean(xc * xc, -1, keepdims=True)
    return xc * jax.lax.rsqrt(v + 1e-5) * g + b


def _bf(x):
    return x.astype(jnp.bfloat16)


def _dot(a, w):
    return jnp.dot(a, w[...], preferred_element_type=jnp.float32)


def _layer_body(lens_ref, x_ref, pe_ref,
                w1h, b1h, w2h, b2h,
                wq1, bq1, wqc, bqc,
                wk1, bk1, wkc, bkc,
                wv1, bv1, wvc, bvc,
                wa, ba, wf1, bf1, wf2, bf2,
                g_ref, beta_ref,
                out_ref, h_s, q_s, k_s, v_s, *, tr):
    b = pl.program_id(0)
    seqlen = lens_ref[b]
    for r in range(0, T, tr):
        sl = slice(r, r + tr)
        x_t = _bf(x_ref[0, sl, :])
        t1 = _bf(jnp.tanh(_dot(x_t, w1h) + b1h[...]))
        h_t = _dot(t1, w2h) + b2h[...] + pe_ref[sl, :]
        h_s[sl, :] = h_t
        hb = _bf(h_t)
        onec = _bf(jnp.where(
            jax.lax.broadcasted_iota(jnp.int32, (tr, E), 1) == 0, 1.0, 0.0))
        for w1, b1, wc, bc, dd in ((wq1, bq1, wqc, bqc, q_s),
                                   (wk1, bk1, wkc, bkc, k_s),
                                   (wv1, bv1, wvc, bvc, v_s)):
            u1 = _bf(jnp.tanh(_dot(hb, w1) + b1[...]))
            pr = _bf(_dot(u1, wc) + bc[...])
            for hh in range(H):
                sl_h = pr[:, hh * E:(hh + 1) * E]
                if dd is v_s:
                    dd[hh, sl, :] = jnp.concatenate([sl_h, onec], axis=1)
                else:
                    dd[hh, sl, :] = sl_h
    madd = jnp.where(
        jax.lax.broadcasted_iota(jnp.int32, (1, T), 1) >= seqlen, _NEG, 0.0)
    for r in range(0, T, tr):
        sl = slice(r, r + tr)
        atts = []
        for hh in range(H):
            qh = q_s[hh, sl, :]
            kh = k_s[hh]
            s = jax.lax.dot_general(
                qh, kh, (((1,), (1,)), ((), ())),
                preferred_element_type=jnp.float32)
            mx = jnp.max(s, axis=-1, keepdims=True)
            p = jnp.exp2(_bf(s - mx + madd))
            aug = _dot(p, v_s.at[hh])
            att_h = aug[:, :E] / aug[:, E:E + 1]
            atts.append(_bf(att_h))
        att = jnp.concatenate(atts, axis=-1)
        acc = _dot(att, wa) + ba[...]
        x2 = _ln(h_s[sl, :] + acc, g_ref[...], beta_ref[...])
        f1 = _bf(jnp.maximum(_dot(_bf(x2), wf1) + bf1[...], 0.0))
        f2 = _dot(f1, wf2) + bf2[...]
        out_ref[0, sl, :] = _ln(x2 + f2, g_ref[...], beta_ref[...])


def _head_body(lens_ref, x_ref,
               wf1, bf1, wfc, bfc, ww1, bw1, wwc, bwc, wlast, blast,
               out_ref, feat_s, logit_s, *, tr):
    b = pl.program_id(0)
    seqlen = lens_ref[b]
    mx = jnp.full((1, D), _NEG, jnp.float32)
    for r in range(0, T, tr):
        sl = slice(r, r + tr)
        hb = _bf(x_ref[0, sl, :])
        u1 = _bf(jnp.tanh(_dot(hb, wf1) + bf1[...]))
        feat_s[sl, :] = _dot(u1, wfc) + bfc[...]
        u2 = _bf(jnp.tanh(_dot(hb, ww1) + bw1[...]))
        lg = _dot(u2, wwc) + bwc[...]
        lg = jnp.where(
            jax.lax.broadcasted_iota(jnp.int32, (tr, D), 0) + r >= seqlen,
            _NEG, lg)
        logit_s[sl, :] = lg
        mx = jnp.maximum(mx, jnp.max(lg, axis=0, keepdims=True))
    num = jnp.zeros((1, D), jnp.float32)
    den = jnp.zeros((1, D), jnp.float32)
    for r in range(0, T, tr):
        sl = slice(r, r + tr)
        p = jnp.exp2(logit_s[sl, :] - mx)
        num = num + jnp.sum(p * feat_s[sl, :], axis=0, keepdims=True)
        den = den + jnp.sum(p, axis=0, keepdims=True)
    pooled = num / den
    out_ref[0] = _dot(_bf(pooled), wlast) + blast[...]


def _lin_w(p):
    return p["w"].T


def _fold(mlp, split_w2d, split_b):
    ws = split_w2d.T
    wc = mlp["l2"]["w"].T @ ws
    bc = mlp["l2"]["b"][None, :] @ ws + split_b[None, :]
    return wc, bc


def _prep_layer(p, scale_q):
    th, ff = p["to_hidden"], p["ff"]
    wqc, bqc = _fold(p["q_mlp"], p["q_split"]["w"].reshape(H * E, D),
                     p["q_split"]["b"].reshape(H * E))
    wkc, bkc = _fold(p["k_mlp"], p["k_split"]["w"].reshape(H * E, D),
                     p["k_split"]["b"].reshape(H * E))
    wvc, bvc = _fold(p["v_mlp"], p["v_split"]["w"].reshape(H * E, D),
                     p["v_split"]["b"].reshape(H * E))
    wqc, bqc = wqc * (scale_q * _LOG2E), bqc * (scale_q * _LOG2E)
    return [
        _bf(_lin_w(th["l1"])), th["l1"]["b"][None],
        _bf(_lin_w(th["l2"])), th["l2"]["b"][None],
        _bf(_lin_w(p["q_mlp"]["l1"])), p["q_mlp"]["l1"]["b"][None],
        _bf(wqc), bqc,
        _bf(_lin_w(p["k_mlp"]["l1"])), p["k_mlp"]["l1"]["b"][None],
        _bf(wkc), bkc,
        _bf(_lin_w(p["v_mlp"]["l1"])), p["v_mlp"]["l1"]["b"][None],
        _bf(wvc), bvc,
        _bf(_lin_w(p["h2a"])), p["h2a"]["b"][None],
        _bf(_lin_w(ff["l1"])), ff["l1"]["b"][None],
        _bf(_lin_w(ff["l2"])), ff["l2"]["b"][None],
        p["ln_g"][None], p["ln_b"][None],
    ]


def _prep_head(params):
    wfc, bfc = _fold(params["ff_mlp"],
                     params["ff_split"]["w"].reshape(H * E, D),
                     params["ff_split"]["b"].reshape(H * E))
    wwc_s, bwc_s = _fold(params["fw_mlp"], params["fw_split"]["w"][:, 0, :],
                         params["fw_split"]["b"][:, 0])
    wwc = jnp.repeat(wwc_s * _LOG2E, E, axis=1)
    bwc = jnp.repeat(bwc_s * _LOG2E, E, axis=1)
    return [
        _bf(_lin_w(params["ff_mlp"]["l1"])), params["ff_mlp"]["l1"]["b"][None],
        _bf(wfc), bfc,
        _bf(_lin_w(params["fw_mlp"]["l1"])), params["fw_mlp"]["l1"]["b"][None],
        _bf(wwc), bwc,
        _bf(_lin_w(params["last"])), params["last"]["b"][None],
    ]


def _full_spec(shape):
    nd = len(shape)
    return pl.BlockSpec(shape, lambda b: (0,) * nd)


_CPARAMS = pltpu.CompilerParams(
    dimension_semantics=("parallel",),
    vmem_limit_bytes=100 * 1024 * 1024,
)


def _layer_call(x, pe, lens, wargs, ci, tr, interpret):
    in_specs = [pl.BlockSpec(memory_space=pltpu.SMEM),
                pl.BlockSpec((1, T, ci), lambda b: (b, 0, 0)),
                _full_spec((T, D))]
    in_specs += [_full_spec(w.shape) for w in wargs]
    return pl.pallas_call(
        functools.partial(_layer_body, tr=tr),
        out_shape=jax.ShapeDtypeStruct((B, T, D), jnp.float32),
        grid=(B,),
        in_specs=in_specs,
        out_specs=pl.BlockSpec((1, T, D), lambda b: (b, 0, 0)),
        scratch_shapes=[
            pltpu.VMEM((T, D), jnp.float32),
            pltpu.VMEM((H, T, E), jnp.bfloat16),
            pltpu.VMEM((H, T, E), jnp.bfloat16),
            pltpu.VMEM((H, T, 2 * E), jnp.bfloat16),
        ],
        compiler_params=_CPARAMS,
        name="enc_layer",
        interpret=interpret,
    )(lens, x, pe, *wargs)


def _head_call(x, lens, wargs, tr, interpret):
    in_specs = [pl.BlockSpec(memory_space=pltpu.SMEM),
                pl.BlockSpec((1, T, D), lambda b: (b, 0, 0))]
    in_specs += [_full_spec(w.shape) for w in wargs]
    return pl.pallas_call(
        functools.partial(_head_body, tr=tr),
        out_shape=jax.ShapeDtypeStruct((B, 1, D), jnp.float32),
        grid=(B,),
        in_specs=in_specs,
        out_specs=pl.BlockSpec((1, 1, D), lambda b: (b, 0, 0)),
        scratch_shapes=[
            pltpu.VMEM((T, D), jnp.float32),
            pltpu.VMEM((T, D), jnp.float32),
        ],
        compiler_params=_CPARAMS,
        name="attn_pool_head",
        interpret=interpret,
    )(lens, x, *wargs)


def kernel(x, lengths, params, interpret=False, tr=1024):
    pe = jnp.asarray(_PE)
    lens = lengths.astype(jnp.int32)
    h = x
    for li, p in enumerate(params["layers"]):
        ci = IN if li == 0 else D
        wargs = _prep_layer(p, E ** -0.5)
        h = _layer_call(h, pe, lens, wargs, ci, tr, interpret)
    hargs = _prep_head(params)
    return _head_call(h, lens, hargs, tr, interpret).reshape(B, D)


# speedup vs baseline: 1.8373x; 1.0062x over previous
import functools

import jax
import jax.numpy as jnp
import numpy as np
from jax.experimental import pallas as pl
from jax.experimental.pallas import tpu as pltpu

B, T, IN = 8, 1024, 80
D, M, H, L = 512, 512, 8, 2
E = D // H

_NEG = -1e30
_LOG2E = 1.4426950408889634


def _pe_table(t, d):
    inv = 10000.0 ** np.arange(0.0, 1.0, 2.0 / d, dtype=np.float32)
    ang = np.arange(t, dtype=np.float32)[:, None] / inv[None, :]
    return np.stack([np.sin(ang), np.cos(ang)], -1).reshape(t, d)


_PE = _pe_table(T, D).astype(np.float32)


def _ln(x, g, b):
    m = jnp.mean(x, -1, keepdims=True)
    xc = x - m
    v = jnp.mean(xc * xc, -1, keepdims=True)
    return xc * jax.lax.rsqrt(v + 1e-5) * g + b


def _bf(x):
    return x.astype(jnp.bfloat16)


def _dot(a, w):
    return jnp.dot(a, w[...], preferred_element_type=jnp.float32)


def _layer_body(lens_ref, x_ref, pe_ref,
                w1h, b1h, w2h, b2h,
                wq1, bq1, wqc, bqc,
                wk1, bk1, wkc, bkc,
                wv1, bv1, wvc, bvc,
                wa, ba, wf1, bf1, wf2, bf2,
                g_ref, beta_ref,
                out_ref, h_s, q_s, k_s, v_s, *, tr):
    b = pl.program_id(0)
    j = pl.program_id(1)
    seqlen = lens_ref[b]

    @pl.when(j == 0)
    def _stage_a():
        for r in range(0, T, tr):
            sl = slice(r, r + tr)
            x_t = _bf(x_ref[0, sl, :])
            t1 = _bf(jnp.tanh(_dot(x_t, w1h) + b1h[...]))
            h_t = _dot(t1, w2h) + b2h[...] + pe_ref[sl, :]
            h_s[sl, :] = h_t
            hb = _bf(h_t)
            onec = _bf(jnp.where(
                jax.lax.broadcasted_iota(jnp.int32, (tr, E), 1) == 0,
                1.0, 0.0))
            for w1, b1, wc, bc, dd in ((wq1, bq1, wqc, bqc, q_s),
                                       (wk1, bk1, wkc, bkc, k_s),
                                       (wv1, bv1, wvc, bvc, v_s)):
                u1 = _bf(jnp.tanh(_dot(hb, w1) + b1[...]))
                pr = _bf(_dot(u1, wc) + bc[...])
                for hh in range(H):
                    sl_h = pr[:, hh * E:(hh + 1) * E]
                    if dd is v_s:
                        dd[hh, sl, :] = jnp.concatenate([sl_h, onec], axis=1)
                    else:
                        dd[hh, sl, :] = sl_h

    @pl.when(j == 1)
    def _stage_b():
        madd = jnp.where(
            jax.lax.broadcasted_iota(jnp.int32, (1, T), 1) >= seqlen,
            _NEG, 0.0)
        for r in range(0, T, tr):
            sl = slice(r, r + tr)
            atts = []
            for hh in range(H):
                qh = q_s[hh, sl, :]
                kh = k_s[hh]
                s = jax.lax.dot_general(
                    qh, kh, (((1,), (1,)), ((), ())),
                    preferred_element_type=jnp.float32)
                mx = jnp.max(s, axis=-1, keepdims=True)
                p = jnp.exp2(_bf(s - mx + madd))
                aug = _dot(p, v_s.at[hh])
                att_h = aug[:, :E] / aug[:, E:E + 1]
                atts.append(_bf(att_h))
            att = jnp.concatenate(atts, axis=-1)
            acc = _dot(att, wa) + ba[...]
            x2 = _ln(h_s[sl, :] + acc, g_ref[...], beta_ref[...])
            f1 = _bf(jnp.maximum(_dot(_bf(x2), wf1) + bf1[...], 0.0))
            f2 = _dot(f1, wf2) + bf2[...]
            out_ref[0, sl, :] = _ln(x2 + f2, g_ref[...], beta_ref[...])


def _head_body(lens_ref, x_ref,
               wf1, bf1, wfc, bfc, ww1, bw1, wwc, bwc, wlast, blast,
               out_ref, feat_s, logit_s, *, tr):
    b = pl.program_id(0)
    seqlen = lens_ref[b]
    mx = jnp.full((1, D), _NEG, jnp.float32)
    for r in range(0, T, tr):
        sl = slice(r, r + tr)
        hb = _bf(x_ref[0, sl, :])
        u1 = _bf(jnp.tanh(_dot(hb, wf1) + bf1[...]))
        feat_s[sl, :] = _dot(u1, wfc) + bfc[...]
        u2 = _bf(jnp.tanh(_dot(hb, ww1) + bw1[...]))
        lg = _dot(u2, wwc) + bwc[...]
        lg = jnp.where(
            jax.lax.broadcasted_iota(jnp.int32, (tr, D), 0) + r >= seqlen,
            _NEG, lg)
        logit_s[sl, :] = lg
        mx = jnp.maximum(mx, jnp.max(lg, axis=0, keepdims=True))
    num = jnp.zeros((1, D), jnp.float32)
    den = jnp.zeros((1, D), jnp.float32)
    for r in range(0, T, tr):
        sl = slice(r, r + tr)
        p = jnp.exp2(logit_s[sl, :] - mx)
        num = num + jnp.sum(p * feat_s[sl, :], axis=0, keepdims=True)
        den = den + jnp.sum(p, axis=0, keepdims=True)
    pooled = num / den
    out_ref[0] = _dot(_bf(pooled), wlast) + blast[...]


def _lin_w(p):
    return p["w"].T


def _fold(mlp, split_w2d, split_b):
    ws = split_w2d.T
    wc = mlp["l2"]["w"].T @ ws
    bc = mlp["l2"]["b"][None, :] @ ws + split_b[None, :]
    return wc, bc


def _prep_layer(p, scale_q):
    th, ff = p["to_hidden"], p["ff"]
    wqc, bqc = _fold(p["q_mlp"], p["q_split"]["w"].reshape(H * E, D),
                     p["q_split"]["b"].reshape(H * E))
    wkc, bkc = _fold(p["k_mlp"], p["k_split"]["w"].reshape(H * E, D),
                     p["k_split"]["b"].reshape(H * E))
    wvc, bvc = _fold(p["v_mlp"], p["v_split"]["w"].reshape(H * E, D),
                     p["v_split"]["b"].reshape(H * E))
    wqc, bqc = wqc * (scale_q * _LOG2E), bqc * (scale_q * _LOG2E)
    return [
        _bf(_lin_w(th["l1"])), th["l1"]["b"][None],
        _bf(_lin_w(th["l2"])), th["l2"]["b"][None],
        _bf(_lin_w(p["q_mlp"]["l1"])), p["q_mlp"]["l1"]["b"][None],
        _bf(wqc), bqc,
        _bf(_lin_w(p["k_mlp"]["l1"])), p["k_mlp"]["l1"]["b"][None],
        _bf(wkc), bkc,
        _bf(_lin_w(p["v_mlp"]["l1"])), p["v_mlp"]["l1"]["b"][None],
        _bf(wvc), bvc,
        _bf(_lin_w(p["h2a"])), p["h2a"]["b"][None],
        _bf(_lin_w(ff["l1"])), ff["l1"]["b"][None],
        _bf(_lin_w(ff["l2"])), ff["l2"]["b"][None],
        p["ln_g"][None], p["ln_b"][None],
    ]


def _prep_head(params):
    wfc, bfc = _fold(params["ff_mlp"],
                     params["ff_split"]["w"].reshape(H * E, D),
                     params["ff_split"]["b"].reshape(H * E))
    wwc_s, bwc_s = _fold(params["fw_mlp"], params["fw_split"]["w"][:, 0, :],
                         params["fw_split"]["b"][:, 0])
    wwc = jnp.repeat(wwc_s * _LOG2E, E, axis=1)
    bwc = jnp.repeat(bwc_s * _LOG2E, E, axis=1)
    return [
        _bf(_lin_w(params["ff_mlp"]["l1"])), params["ff_mlp"]["l1"]["b"][None],
        _bf(wfc), bfc,
        _bf(_lin_w(params["fw_mlp"]["l1"])), params["fw_mlp"]["l1"]["b"][None],
        _bf(wwc), bwc,
        _bf(_lin_w(params["last"])), params["last"]["b"][None],
    ]


def _full_spec(shape):
    nd = len(shape)
    return pl.BlockSpec(shape, lambda *_: (0,) * nd)


_CPARAMS = pltpu.CompilerParams(
    dimension_semantics=("parallel", "arbitrary"),
    vmem_limit_bytes=100 * 1024 * 1024,
)

_CPARAMS_1D = pltpu.CompilerParams(
    dimension_semantics=("parallel",),
    vmem_limit_bytes=100 * 1024 * 1024,
)


def _layer_call(x, pe, lens, wargs, ci, tr, interpret):
    in_specs = [pl.BlockSpec(memory_space=pltpu.SMEM),
                pl.BlockSpec((1, T, ci), lambda b, j: (b, 0, 0)),
                _full_spec((T, D))]
    in_specs += [_full_spec(w.shape) for w in wargs]
    return pl.pallas_call(
        functools.partial(_layer_body, tr=tr),
        out_shape=jax.ShapeDtypeStruct((B, T, D), jnp.float32),
        grid=(B, 2),
        in_specs=in_specs,
        out_specs=pl.BlockSpec((1, T, D), lambda b, j: (b, 0, 0)),
        scratch_shapes=[
            pltpu.VMEM((T, D), jnp.float32),
            pltpu.VMEM((H, T, E), jnp.bfloat16),
            pltpu.VMEM((H, T, E), jnp.bfloat16),
            pltpu.VMEM((H, T, 2 * E), jnp.bfloat16),
        ],
        compiler_params=_CPARAMS,
        name="enc_layer",
        interpret=interpret,
    )(lens, x, pe, *wargs)


def _head_call(x, lens, wargs, tr, interpret):
    in_specs = [pl.BlockSpec(memory_space=pltpu.SMEM),
                pl.BlockSpec((1, T, D), lambda b: (b, 0, 0))]
    in_specs += [_full_spec(w.shape) for w in wargs]
    return pl.pallas_call(
        functools.partial(_head_body, tr=tr),
        out_shape=jax.ShapeDtypeStruct((B, 1, D), jnp.float32),
        grid=(B,),
        in_specs=in_specs,
        out_specs=pl.BlockSpec((1, 1, D), lambda b: (b, 0, 0)),
        scratch_shapes=[
            pltpu.VMEM((T, D), jnp.float32),
            pltpu.VMEM((T, D), jnp.float32),
        ],
        compiler_params=_CPARAMS_1D,
        name="attn_pool_head",
        interpret=interpret,
    )(lens, x, *wargs)


def kernel(x, lengths, params, interpret=False, tr=1024):
    pe = jnp.asarray(_PE)
    lens = lengths.astype(jnp.int32)
    h = x
    for li, p in enumerate(params["layers"]):
        ci = IN if li == 0 else D
        wargs = _prep_layer(p, E ** -0.5)
        h = _layer_call(h, pe, lens, wargs, ci, tr, interpret)
    hargs = _prep_head(params)
    return _head_call(h, lens, hargs, tr, interpret).reshape(B, D)


# speedup vs baseline: 1.8695x; 1.0176x over previous
import functools

import jax
import jax.numpy as jnp
import numpy as np
from jax.experimental import pallas as pl
from jax.experimental.pallas import tpu as pltpu

B, T, IN = 8, 1024, 80
D, M, H, L = 512, 512, 8, 2
E = D // H

_NEG = -1e30
_LOG2E = 1.4426950408889634


def _pe_table(t, d):
    inv = 10000.0 ** np.arange(0.0, 1.0, 2.0 / d, dtype=np.float32)
    ang = np.arange(t, dtype=np.float32)[:, None] / inv[None, :]
    return np.stack([np.sin(ang), np.cos(ang)], -1).reshape(t, d)


_PE = _pe_table(T, D).astype(np.float32)


def _ln(x, g, b):
    m = jnp.mean(x, -1, keepdims=True)
    xc = x - m
    v = jnp.mean(xc * xc, -1, keepdims=True)
    return xc * jax.lax.rsqrt(v + 1e-5) * g + b


def _bf(x):
    return x.astype(jnp.bfloat16)


def _dot(a, w):
    return jnp.dot(a, w[...], preferred_element_type=jnp.float32)


def _layer_body(lens_ref, x_ref, pe_ref,
                w1h, b1h, w2h, b2h,
                wq1, bq1, wqc, bqc,
                wk1, bk1, wkc, bkc,
                wv1, bv1, wvc, bvc,
                wa, ba, wf1, bf1, wf2, bf2,
                g_ref, beta_ref,
                out_ref, h_s, q_s, k_s, v_s, *, tr):
    b = pl.program_id(0)
    j = pl.program_id(1)
    seqlen = lens_ref[b]

    @pl.when(j == 0)
    def _stage_a():
        for r in range(0, T, tr):
            sl = slice(r, r + tr)
            x_t = _bf(x_ref[0, sl, :])
            t1 = _bf(jnp.tanh(_dot(x_t, w1h) + b1h[...]))
            h_t = _dot(t1, w2h) + b2h[...] + pe_ref[sl, :]
            h_s[sl, :] = h_t
            hb = _bf(h_t)
            onec = _bf(jnp.where(
                jax.lax.broadcasted_iota(jnp.int32, (tr, E), 1) == 0,
                1.0, 0.0))
            for w1, b1, wc, bc, dd in ((wq1, bq1, wqc, bqc, q_s),
                                       (wk1, bk1, wkc, bkc, k_s),
                                       (wv1, bv1, wvc, bvc, v_s)):
                u1 = _bf(jnp.tanh(_dot(hb, w1) + b1[...]))
                pr = _bf(_dot(u1, wc) + bc[...])
                for hh in range(H):
                    sl_h = pr[:, hh * E:(hh + 1) * E]
                    if dd is v_s:
                        dd[hh, sl, :] = jnp.concatenate([sl_h, onec], axis=1)
                    else:
                        dd[hh, sl, :] = sl_h

    @pl.when(j == 1)
    def _stage_b():
        madd = jnp.where(
            jax.lax.broadcasted_iota(jnp.int32, (1, T), 1) >= seqlen,
            _NEG, 0.0)
        for r in range(0, T, tr):
            sl = slice(r, r + tr)
            atts = []
            for hh in range(H):
                qh = q_s[hh, sl, :]
                kh = k_s[hh]
                s = jax.lax.dot_general(
                    qh, kh, (((1,), (1,)), ((), ())),
                    preferred_element_type=jnp.float32)
                mx = jnp.max(s, axis=-1, keepdims=True)
                p = _bf(jnp.exp2(s - mx + madd))
                aug = _dot(p, v_s.at[hh])
                att_h = aug[:, :E] / aug[:, E:E + 1]
                atts.append(_bf(att_h))
            att = jnp.concatenate(atts, axis=-1)
            acc = _dot(att, wa) + ba[...]
            x2 = _ln(h_s[sl, :] + acc, g_ref[...], beta_ref[...])
            f1 = _bf(jnp.maximum(_dot(_bf(x2), wf1) + bf1[...], 0.0))
            f2 = _dot(f1, wf2) + bf2[...]
            out_ref[0, sl, :] = _ln(x2 + f2, g_ref[...], beta_ref[...])


def _head_body(lens_ref, x_ref,
               wf1, bf1, wfc, bfc, ww1, bw1, wwc, bwc, wlast, blast,
               out_ref, feat_s, logit_s, *, tr):
    b = pl.program_id(0)
    seqlen = lens_ref[b]
    mx = jnp.full((1, D), _NEG, jnp.float32)
    for r in range(0, T, tr):
        sl = slice(r, r + tr)
        hb = _bf(x_ref[0, sl, :])
        u1 = _bf(jnp.tanh(_dot(hb, wf1) + bf1[...]))
        feat_s[sl, :] = _dot(u1, wfc) + bfc[...]
        u2 = _bf(jnp.tanh(_dot(hb, ww1) + bw1[...]))
        lg = _dot(u2, wwc) + bwc[...]
        lg = jnp.where(
            jax.lax.broadcasted_iota(jnp.int32, (tr, D), 0) + r >= seqlen,
            _NEG, lg)
        logit_s[sl, :] = lg
        mx = jnp.maximum(mx, jnp.max(lg, axis=0, keepdims=True))
    num = jnp.zeros((1, D), jnp.float32)
    den = jnp.zeros((1, D), jnp.float32)
    for r in range(0, T, tr):
        sl = slice(r, r + tr)
        p = jnp.exp2(logit_s[sl, :] - mx)
        num = num + jnp.sum(p * feat_s[sl, :], axis=0, keepdims=True)
        den = den + jnp.sum(p, axis=0, keepdims=True)
    pooled = num / den
    out_ref[0] = _dot(_bf(pooled), wlast) + blast[...]


def _lin_w(p):
    return p["w"].T


def _fold(mlp, split_w2d, split_b):
    ws = split_w2d.T
    wc = mlp["l2"]["w"].T @ ws
    bc = mlp["l2"]["b"][None, :] @ ws + split_b[None, :]
    return wc, bc


def _prep_layer(p, scale_q):
    th, ff = p["to_hidden"], p["ff"]
    wqc, bqc = _fold(p["q_mlp"], p["q_split"]["w"].reshape(H * E, D),
                     p["q_split"]["b"].reshape(H * E))
    wkc, bkc = _fold(p["k_mlp"], p["k_split"]["w"].reshape(H * E, D),
                     p["k_split"]["b"].reshape(H * E))
    wvc, bvc = _fold(p["v_mlp"], p["v_split"]["w"].reshape(H * E, D),
                     p["v_split"]["b"].reshape(H * E))
    wqc, bqc = wqc * (scale_q * _LOG2E), bqc * (scale_q * _LOG2E)
    return [
        _bf(_lin_w(th["l1"])), th["l1"]["b"][None],
        _bf(_lin_w(th["l2"])), th["l2"]["b"][None],
        _bf(_lin_w(p["q_mlp"]["l1"])), p["q_mlp"]["l1"]["b"][None],
        _bf(wqc), bqc,
        _bf(_lin_w(p["k_mlp"]["l1"])), p["k_mlp"]["l1"]["b"][None],
        _bf(wkc), bkc,
        _bf(_lin_w(p["v_mlp"]["l1"])), p["v_mlp"]["l1"]["b"][None],
        _bf(wvc), bvc,
        _bf(_lin_w(p["h2a"])), p["h2a"]["b"][None],
        _bf(_lin_w(ff["l1"])), ff["l1"]["b"][None],
        _bf(_lin_w(ff["l2"])), ff["l2"]["b"][None],
        p["ln_g"][None], p["ln_b"][None],
    ]


def _prep_head(params):
    wfc, bfc = _fold(params["ff_mlp"],
                     params["ff_split"]["w"].reshape(H * E, D),
                     params["ff_split"]["b"].reshape(H * E))
    wwc_s, bwc_s = _fold(params["fw_mlp"], params["fw_split"]["w"][:, 0, :],
                         params["fw_split"]["b"][:, 0])
    wwc = jnp.repeat(wwc_s * _LOG2E, E, axis=1)
    bwc = jnp.repeat(bwc_s * _LOG2E, E, axis=1)
    return [
        _bf(_lin_w(params["ff_mlp"]["l1"])), params["ff_mlp"]["l1"]["b"][None],
        _bf(wfc), bfc,
        _bf(_lin_w(params["fw_mlp"]["l1"])), params["fw_mlp"]["l1"]["b"][None],
        _bf(wwc), bwc,
        _bf(_lin_w(params["last"])), params["last"]["b"][None],
    ]


def _full_spec(shape):
    nd = len(shape)
    return pl.BlockSpec(shape, lambda *_: (0,) * nd)


_CPARAMS = pltpu.CompilerParams(
    dimension_semantics=("parallel", "arbitrary"),
    vmem_limit_bytes=100 * 1024 * 1024,
)

_CPARAMS_1D = pltpu.CompilerParams(
    dimension_semantics=("parallel",),
    vmem_limit_bytes=100 * 1024 * 1024,
)


def _layer_call(x, pe, lens, wargs, ci, tr, interpret):
    in_specs = [pl.BlockSpec(memory_space=pltpu.SMEM),
                pl.BlockSpec((1, T, ci), lambda b, j: (b, 0, 0)),
                _full_spec((T, D))]
    in_specs += [_full_spec(w.shape) for w in wargs]
    return pl.pallas_call(
        functools.partial(_layer_body, tr=tr),
        out_shape=jax.ShapeDtypeStruct((B, T, D), jnp.float32),
        grid=(B, 2),
        in_specs=in_specs,
        out_specs=pl.BlockSpec((1, T, D), lambda b, j: (b, 0, 0)),
        scratch_shapes=[
            pltpu.VMEM((T, D), jnp.float32),
            pltpu.VMEM((H, T, E), jnp.bfloat16),
            pltpu.VMEM((H, T, E), jnp.bfloat16),
            pltpu.VMEM((H, T, 2 * E), jnp.bfloat16),
        ],
        compiler_params=_CPARAMS,
        name="enc_layer",
        interpret=interpret,
    )(lens, x, pe, *wargs)


def _head_call(x, lens, wargs, tr, interpret):
    in_specs = [pl.BlockSpec(memory_space=pltpu.SMEM),
                pl.BlockSpec((1, T, D), lambda b: (b, 0, 0))]
    in_specs += [_full_spec(w.shape) for w in wargs]
    return pl.pallas_call(
        functools.partial(_head_body, tr=tr),
        out_shape=jax.ShapeDtypeStruct((B, 1, D), jnp.float32),
        grid=(B,),
        in_specs=in_specs,
        out_specs=pl.BlockSpec((1, 1, D), lambda b: (b, 0, 0)),
        scratch_shapes=[
            pltpu.VMEM((T, D), jnp.float32),
            pltpu.VMEM((T, D), jnp.float32),
        ],
        compiler_params=_CPARAMS_1D,
        name="attn_pool_head",
        interpret=interpret,
    )(lens, x, *wargs)


def kernel(x, lengths, params, interpret=False, tr=1024):
    pe = jnp.asarray(_PE)
    lens = lengths.astype(jnp.int32)
    h = x
    for li, p in enumerate(params["layers"]):
        ci = IN if li == 0 else D
        wargs = _prep_layer(p, E ** -0.5)
        h = _layer_call(h, pe, lens, wargs, ci, tr, interpret)
    hargs = _prep_head(params)
    return _head_call(h, lens, hargs, tr, interpret).reshape(B, D)
